# baseline (speedup 1.0000x reference)
import functools

import jax
import jax.numpy as jnp
from jax import lax
from jax.experimental import pallas as pl
from jax.experimental.pallas import tpu as pltpu
from jax.experimental.pallas import tpu_sc as plsc

B = 16384
H = 64

_info = plsc.get_sparse_core_info()
_NC = _info.num_cores
_NS = _info.num_subcores
_NW = _NC * _NS
_BPW = B // _NW
_CH = 2
_BPC = _BPW // _CH


def _sc_gather_body(tab_p, tab_n, idx_p_hbm, idx_n_hbm, out_p, out_n,
                    idxp_v, idxn_v, rowsp_v, rowsn_v, semp, semn):
    wid = lax.axis_index("s") * _NC + lax.axis_index("c")
    base = wid * _BPW
    pltpu.sync_copy(idx_p_hbm.at[pl.ds(base, _BPW)], idxp_v)
    pltpu.sync_copy(idx_n_hbm.at[pl.ds(base, _BPW)], idxn_v)
    for c in range(_CH):
        off = c * _BPC
        cp = pltpu.async_copy(tab_p.at[idxp_v.at[pl.ds(off, _BPC)]],
                              rowsp_v, semp)
        cn = pltpu.async_copy(tab_n.at[idxn_v.at[pl.ds(off, _BPC)]],
                              rowsn_v, semn)
        cp.wait()
        cn.wait()
        pltpu.sync_copy(rowsp_v, out_p.at[pl.ds(base + off, _BPC)])
        pltpu.sync_copy(rowsn_v, out_n.at[pl.ds(base + off, _BPC)])


_sc_gather = functools.partial(
    pl.kernel,
    mesh=plsc.VectorSubcoreMesh(core_axis_name="c", subcore_axis_name="s"),
    out_type=[
        jax.ShapeDtypeStruct((B, 2 * H), jnp.float32),
        jax.ShapeDtypeStruct((B, 2 * H), jnp.float32),
    ],
    scratch_types=[
        pltpu.VMEM((_BPW,), jnp.int32),
        pltpu.VMEM((_BPW,), jnp.int32),
        pltpu.VMEM((_BPC, 2 * H), jnp.float32),
        pltpu.VMEM((_BPC, 2 * H), jnp.float32),
        pltpu.SemaphoreType.DMA,
        pltpu.SemaphoreType.DMA,
    ],
)(_sc_gather_body)


_BM = 2048


def _mlp_body(gp_ref, gn_ref, pp_ref, pn_ref, w1a_ref, w1b_ref, b1_ref,
              w2_ref, b2_ref, w3_ref, b3_ref, o_ref):
    dot = functools.partial(jnp.dot, preferred_element_type=jnp.float32,
                            precision=lax.Precision.HIGHEST)
    gp = gp_ref[...]
    gn = gn_ref[...]
    p = jnp.where(pp_ref[...] == 1, gp[:, H:], gp[:, :H])
    n = jnp.where(pn_ref[...] == 1, gn[:, H:], gn[:, :H])
    h = dot(p, w1a_ref[...]) + dot(n, w1b_ref[...])
    h = jnp.maximum(h + b1_ref[...], 0.0)
    h = jnp.maximum(dot(h, w2_ref[...]) + b2_ref[...], 0.0)
    o_ref[...] = dot(h, w3_ref[...]) + b3_ref[...]


def _mlp(gp, gn, pp, pn, w1a, w1b, b1, w2, b2, w3, b3):
    grid = (B // _BM,)
    return pl.pallas_call(
        _mlp_body,
        grid=grid,
        in_specs=[
            pl.BlockSpec((_BM, 2 * H), lambda i: (i, 0)),
            pl.BlockSpec((_BM, 2 * H), lambda i: (i, 0)),
            pl.BlockSpec((_BM, 1), lambda i: (i, 0)),
            pl.BlockSpec((_BM, 1), lambda i: (i, 0)),
            pl.BlockSpec((H, H), lambda i: (0, 0)),
            pl.BlockSpec((H, H), lambda i: (0, 0)),
            pl.BlockSpec((1, H), lambda i: (0, 0)),
            pl.BlockSpec((H, H), lambda i: (0, 0)),
            pl.BlockSpec((1, H), lambda i: (0, 0)),
            pl.BlockSpec((H, 1), lambda i: (0, 0)),
            pl.BlockSpec((1, 1), lambda i: (0, 0)),
        ],
        out_specs=pl.BlockSpec((_BM, 1), lambda i: (i, 0)),
        out_shape=jax.ShapeDtypeStruct((B, 1), jnp.float32),
    )(gp, gn, pp, pn, w1a, w1b, b1, w2, b2, w3, b3)


def kernel(x, emb_proton, emb_neutron, W1, b1, W2, b2, W3, b3):
    x = x.astype(jnp.int32)
    idx_p = x[:, 0]
    idx_n = x[:, 1]
    tab_p = emb_proton.reshape(-1, 2 * H)
    tab_n = emb_neutron.reshape(-1, 2 * H)
    gp, gn = _sc_gather(tab_p, tab_n, idx_p >> 1, idx_n >> 1)
    pp = (idx_p & 1).reshape(B, 1)
    pn = (idx_n & 1).reshape(B, 1)
    return _mlp(gp, gn, pp, pn, W1[:H], W1[H:], b1.reshape(1, H), W2,
                b2.reshape(1, H), W3, b3.reshape(1, 1))


# speedup vs baseline: 3.2153x; 3.2153x over previous
import functools

import jax
import jax.numpy as jnp
from jax import lax
from jax.experimental import pallas as pl
from jax.experimental.pallas import tpu as pltpu
from jax.experimental.pallas import tpu_sc as plsc

B = 16384
H = 64
V = 100000

_info = plsc.get_sparse_core_info()
_NC = _info.num_cores
_NS = _info.num_subcores
_NW = _NC * _NS
_HPW = H // _NW
_CHUNK = 4096
_NCHUNK = B // _CHUNK


def _sc_gather_body(tabT_p, tabT_n, idx_p_hbm, idx_n_hbm, out_p, out_n,
                    idx_v, slice_v, out_v):
    wid = lax.axis_index("s") * _NC + lax.axis_index("c")
    for tabT, out_hbm, idx_hbm in (
        (tabT_p, out_p, idx_p_hbm),
        (tabT_n, out_n, idx_n_hbm),
    ):
        pltpu.sync_copy(idx_hbm, idx_v)
        for j in range(_HPW):
            h = wid * _HPW + j
            pltpu.sync_copy(tabT.at[h], slice_v)
            for c in range(_NCHUNK):
                base = c * _CHUNK

                @plsc.parallel_loop(0, _CHUNK, 16, unroll=8)
                def _(k):
                    iv = idx_v[pl.ds(base + k, 16)]
                    out_v[pl.ds(k, 16)] = plsc.load_gather(slice_v, [iv])

                pltpu.sync_copy(out_v, out_hbm.at[h, pl.ds(base, _CHUNK)])


_sc_gather = functools.partial(
    pl.kernel,
    mesh=plsc.VectorSubcoreMesh(core_axis_name="c", subcore_axis_name="s"),
    out_type=[
        jax.ShapeDtypeStruct((H, B), jnp.float32),
        jax.ShapeDtypeStruct((H, B), jnp.float32),
    ],
    scratch_types=[
        pltpu.VMEM((B,), jnp.int32),
        pltpu.VMEM((V,), jnp.float32),
        pltpu.VMEM((_CHUNK,), jnp.float32),
    ],
    compiler_params=pltpu.CompilerParams(needs_layout_passes=False),
)(_sc_gather_body)


_BN = 2048


def _mlp_body(pt_ref, nt_ref, w1a_ref, w1b_ref, b1_ref, w2_ref, b2_ref,
              w3_ref, b3_ref, o_ref):
    dot = functools.partial(
        lax.dot_general,
        dimension_numbers=(((0,), (0,)), ((), ())),
        preferred_element_type=jnp.float32,
        precision=lax.Precision.HIGHEST,
    )
    h = dot(w1a_ref[...], pt_ref[...]) + dot(w1b_ref[...], nt_ref[...])
    h = jnp.maximum(h + b1_ref[...], 0.0)
    h = jnp.maximum(dot(w2_ref[...], h) + b2_ref[...], 0.0)
    o_ref[...] = dot(w3_ref[...], h) + b3_ref[...]


def _mlp(pt, nt, w1a, w1b, b1, w2, b2, w3, b3):
    grid = (B // _BN,)
    return pl.pallas_call(
        _mlp_body,
        grid=grid,
        in_specs=[
            pl.BlockSpec((H, _BN), lambda i: (0, i)),
            pl.BlockSpec((H, _BN), lambda i: (0, i)),
            pl.BlockSpec((H, H), lambda i: (0, 0)),
            pl.BlockSpec((H, H), lambda i: (0, 0)),
            pl.BlockSpec((H, 1), lambda i: (0, 0)),
            pl.BlockSpec((H, H), lambda i: (0, 0)),
            pl.BlockSpec((H, 1), lambda i: (0, 0)),
            pl.BlockSpec((H, 1), lambda i: (0, 0)),
            pl.BlockSpec((1, 1), lambda i: (0, 0)),
        ],
        out_specs=pl.BlockSpec((1, _BN), lambda i: (0, i)),
        out_shape=jax.ShapeDtypeStruct((1, B), jnp.float32),
    )(pt, nt, w1a, w1b, b1, w2, b2, w3, b3)


def kernel(x, emb_proton, emb_neutron, W1, b1, W2, b2, W3, b3):
    x = x.astype(jnp.int32)
    idx_p = x[:, 0]
    idx_n = x[:, 1]
    pt, nt = _sc_gather(emb_proton.T, emb_neutron.T, idx_p, idx_n)
    out_t = _mlp(pt, nt, W1[:H], W1[H:], b1.reshape(H, 1), W2,
                 b2.reshape(H, 1), W3, b3.reshape(1, 1))
    return out_t.reshape(B, 1)


# speedup vs baseline: 3.7186x; 1.1565x over previous
import functools

import jax
import jax.numpy as jnp
from jax import lax
from jax.experimental import pallas as pl
from jax.experimental.pallas import tpu as pltpu
from jax.experimental.pallas import tpu_sc as plsc

B = 16384
H = 64
V = 100000

_info = plsc.get_sparse_core_info()
_NC = _info.num_cores
_NS = _info.num_subcores
_NW = _NC * _NS
_HPW = H // _NW
_CHUNK = 4096
_NCHUNK = B // _CHUNK


def _sc_gather_body(tabT_p, tabT_n, idx_p_hbm, idx_n_hbm, out_p, out_n,
                    idx_v, slice_v, out_v):
    wid = lax.axis_index("s") * _NC + lax.axis_index("c")
    for tabT, out_hbm, idx_hbm in (
        (tabT_p, out_p, idx_p_hbm),
        (tabT_n, out_n, idx_n_hbm),
    ):
        pltpu.sync_copy(idx_hbm, idx_v)
        for j in range(_HPW):
            h = wid * _HPW + j
            pltpu.sync_copy(tabT.at[h], slice_v)
            for c in range(_NCHUNK):
                base = c * _CHUNK

                @plsc.parallel_loop(0, _CHUNK, 16, unroll=8)
                def _(k):
                    iv = idx_v[pl.ds(base + k, 16)]
                    out_v[pl.ds(k, 16)] = plsc.load_gather(slice_v, [iv])

                pltpu.sync_copy(out_v, out_hbm.at[h, pl.ds(base, _CHUNK)])


_sc_gather = functools.partial(
    pl.kernel,
    mesh=plsc.VectorSubcoreMesh(core_axis_name="c", subcore_axis_name="s"),
    out_type=[
        jax.ShapeDtypeStruct((H, B), jnp.float32),
        jax.ShapeDtypeStruct((H, B), jnp.float32),
    ],
    scratch_types=[
        pltpu.VMEM((B,), jnp.int32),
        pltpu.VMEM((V,), jnp.float32),
        pltpu.VMEM((_CHUNK,), jnp.float32),
    ],
    compiler_params=pltpu.CompilerParams(needs_layout_passes=False),
)(_sc_gather_body)


_BN = 4096


def _mlp_body(pt_ref, nt_ref, w1a_ref, w1b_ref, b1_ref, w2_ref, b2_ref,
              w3_ref, b3_ref, o_ref):
    dot = functools.partial(
        lax.dot_general,
        dimension_numbers=(((0,), (0,)), ((), ())),
        preferred_element_type=jnp.float32,
    )
    h = dot(w1a_ref[...], pt_ref[...]) + dot(w1b_ref[...], nt_ref[...])
    h = jnp.maximum(h + b1_ref[...], 0.0)
    h = jnp.maximum(dot(w2_ref[...], h) + b2_ref[...], 0.0)
    o_ref[...] = dot(w3_ref[...], h) + b3_ref[...]


def _mlp(pt, nt, w1a, w1b, b1, w2, b2, w3, b3):
    grid = (B // _BN,)
    return pl.pallas_call(
        _mlp_body,
        grid=grid,
        in_specs=[
            pl.BlockSpec((H, _BN), lambda i: (0, i)),
            pl.BlockSpec((H, _BN), lambda i: (0, i)),
            pl.BlockSpec((H, H), lambda i: (0, 0)),
            pl.BlockSpec((H, H), lambda i: (0, 0)),
            pl.BlockSpec((H, 1), lambda i: (0, 0)),
            pl.BlockSpec((H, H), lambda i: (0, 0)),
            pl.BlockSpec((H, 1), lambda i: (0, 0)),
            pl.BlockSpec((H, 1), lambda i: (0, 0)),
            pl.BlockSpec((1, 1), lambda i: (0, 0)),
        ],
        out_specs=pl.BlockSpec((1, _BN), lambda i: (0, i)),
        out_shape=jax.ShapeDtypeStruct((1, B), jnp.float32),
    )(pt, nt, w1a, w1b, b1, w2, b2, w3, b3)


def kernel(x, emb_proton, emb_neutron, W1, b1, W2, b2, W3, b3):
    x = x.astype(jnp.int32)
    idx_p = x[:, 0]
    idx_n = x[:, 1]
    pt, nt = _sc_gather(emb_proton.T, emb_neutron.T, idx_p, idx_n)
    out_t = _mlp(pt, nt, W1[:H], W1[H:], b1.reshape(H, 1), W2,
                 b2.reshape(H, 1), W3, b3.reshape(1, 1))
    return out_t.reshape(B, 1)
